# Initial kernel scaffold; baseline (speedup 1.0000x reference)
import jax
import jax.numpy as jnp
from jax.experimental import pallas as pl


def kernel(xs, mem0, mm1, mm2):
    raise NotImplementedError("write your pallas kernel here")


# speedup vs baseline: 1.0099x; 1.0099x over previous
import functools

import jax
import jax.numpy as jnp
from jax import lax
from jax.experimental import pallas as pl

_RHO = 1e-08


def _fused_body(xs_ref, mem0_ref, mm1_ref, mm2_ref, out_ref):
    f32 = jnp.float32
    B = xs_ref.shape[0]
    FB = xs_ref.shape[1]
    M0 = mem0_ref.shape[1]
    H1 = mm1_ref.shape[2]
    H2 = mm2_ref.shape[2]

    x = xs_ref[...] - 0.5
    xn = jnp.sqrt(jnp.sum(x * x, axis=-1))

    idx1 = []
    val1 = []
    for n in range(4):
        prop = jnp.zeros((H1, B), f32)
        sumsq = jnp.zeros((1, B), f32)
        for c in range(4):
            f = 4 * n + c
            m = mem0_ref[f] - 0.5
            num = lax.dot_general(
                m, x[:, f, :], (((1,), (1,)), ((), ())),
                preferred_element_type=f32) * 0.5
            mn = jnp.sqrt(jnp.sum(m * m, axis=-1))
            h0 = num / (mn[:, None] * xn[:, f][None, :] + _RHO) + 0.5
            vmax = jnp.max(h0, axis=0)
            iota = lax.broadcasted_iota(jnp.int32, (M0, B), 0)
            amax = jnp.min(jnp.where(h0 == vmax[None, :], iota, M0), axis=0)
            sel = jnp.where(iota == amax[None, :], vmax[None, :], 0.0)
            prop = prop + lax.dot_general(
                mm1_ref[n, c], sel, (((1,), (0,)), ((), ())),
                preferred_element_type=f32)
            sumsq = sumsq + (vmax * vmax)[None, :]
        coeff = 1.0 / (4.0 * jnp.sqrt(sumsq) + _RHO)
        h1 = prop * coeff
        v1 = jnp.max(h1, axis=0)
        iota1 = lax.broadcasted_iota(jnp.int32, (H1, B), 0)
        a1 = jnp.min(jnp.where(h1 == v1[None, :], iota1, H1), axis=0)
        idx1.append(a1)
        val1.append(v1)

    prop2 = jnp.zeros((H2, B), f32)
    sumsq2 = jnp.zeros((1, B), f32)
    iota1 = lax.broadcasted_iota(jnp.int32, (H1, B), 0)
    for c in range(4):
        sel2 = jnp.where(iota1 == idx1[c][None, :], val1[c][None, :], 0.0)
        prop2 = prop2 + lax.dot_general(
            mm2_ref[0, c], sel2, (((1,), (0,)), ((), ())),
            preferred_element_type=f32)
        sumsq2 = sumsq2 + (val1[c] * val1[c])[None, :]
    coeff2 = 1.0 / (4.0 * jnp.sqrt(sumsq2) + _RHO)
    out_ref[...] = (prop2 * coeff2)[None, :, :]


@jax.jit
def kernel(xs, mem0, mm1, mm2):
    B, F, D = xs.shape
    M0 = mem0.shape[1]
    N1, C1, H1, _ = mm1.shape
    N2, C2, H2, _ = mm2.shape
    FB = F // N2

    out = pl.pallas_call(
        _fused_body,
        grid=(N2,),
        in_specs=[
            pl.BlockSpec((B, FB, D), lambda i: (0, i, 0)),
            pl.BlockSpec((FB, M0, D), lambda i: (i, 0, 0)),
            pl.BlockSpec((N1 // N2, C1, H1, M0), lambda i: (i, 0, 0, 0)),
            pl.BlockSpec((1, C2, H2, H1), lambda i: (i, 0, 0, 0)),
        ],
        out_specs=pl.BlockSpec((1, H2, B), lambda i: (i, 0, 0)),
        out_shape=jax.ShapeDtypeStruct((N2, H2, B), jnp.float32),
    )(xs, mem0, mm1, mm2)
    return jnp.transpose(out, (2, 0, 1))
